# Initial kernel scaffold; baseline (speedup 1.0000x reference)
import jax
import jax.numpy as jnp
from jax.experimental import pallas as pl


def kernel(inputs, embedding_weight):
    raise NotImplementedError("write your pallas kernel here")


# speedup vs baseline: 9.9424x; 9.9424x over previous
import jax
import jax.numpy as jnp
from jax.experimental import pallas as pl
from jax.experimental.pallas import tpu as pltpu

NUM_CODES = 1024
DIM = 64
COMMIT = 0.25
K = 3
ROWS = 16384
BLOCK = 512
GRID = ROWS // BLOCK


def _vq_body(x_ref, e_ref, enc_ref, q0_ref, q1_ref, q2_ref, loss_ref, perp_ref,
             cnt_ref, acc_ref):
    i = pl.program_id(0)
    x = x_ref[...]
    e = e_ref[...]

    xsq = jnp.sum(x * x, axis=1, keepdims=True)
    esq = jnp.sum(e * e, axis=1)[None, :]
    xe = jax.lax.dot_general(x, e, (((1,), (1,)), ((), ())),
                             preferred_element_type=jnp.float32)
    d = xsq + esq - 2.0 * xe

    iota = jax.lax.broadcasted_iota(jnp.int32, d.shape, 1)
    q_refs = (q0_ref, q1_ref, q2_ref)
    last_oh = None
    for k in range(K):
        dmin = jnp.min(d, axis=1, keepdims=True)
        idx = jnp.min(jnp.where(d == dmin, iota, NUM_CODES), axis=1,
                      keepdims=True)
        oh = (iota == idx).astype(jnp.float32)
        q = jax.lax.dot_general(oh, e, (((1,), (0,)), ((), ())),
                                precision=jax.lax.Precision.HIGHEST,
                                preferred_element_type=jnp.float32)
        q_refs[k][...] = q
        if k == 0:
            part_loss = jnp.sum((q - x) ** 2, keepdims=True)
        if k < K - 1:
            d = jnp.where(oh > 0.0, jnp.inf, d)
        last_oh = oh

    enc_ref[...] = last_oh

    @pl.when(i == 0)
    def _():
        acc_ref[...] = jnp.zeros_like(acc_ref)
        cnt_ref[...] = jnp.zeros_like(cnt_ref)

    acc_ref[...] += part_loss
    cnt_ref[...] += jnp.sum(last_oh, axis=0, keepdims=True)

    @pl.when(i == GRID - 1)
    def _():
        loss_ref[...] = acc_ref[...] * (COMMIT / (ROWS * DIM))
        p = cnt_ref[...] * (1.0 / ROWS)
        perp_ref[...] = jnp.exp(-jnp.sum(p * jnp.log(p + 1e-10),
                                         keepdims=True))


def kernel(inputs, embedding_weight):
    x = jnp.transpose(inputs, (0, 2, 3, 1))
    in_shape = x.shape
    flat = x.reshape(ROWS, DIM)

    out_shapes = (
        jax.ShapeDtypeStruct((ROWS, NUM_CODES), jnp.float32),
        jax.ShapeDtypeStruct((ROWS, DIM), jnp.float32),
        jax.ShapeDtypeStruct((ROWS, DIM), jnp.float32),
        jax.ShapeDtypeStruct((ROWS, DIM), jnp.float32),
        jax.ShapeDtypeStruct((1, 1), jnp.float32),
        jax.ShapeDtypeStruct((1, 1), jnp.float32),
    )
    enc, q0, q1, q2, loss, perp = pl.pallas_call(
        _vq_body,
        grid=(GRID,),
        in_specs=[
            pl.BlockSpec((BLOCK, DIM), lambda i: (i, 0)),
            pl.BlockSpec((NUM_CODES, DIM), lambda i: (0, 0)),
        ],
        out_specs=[
            pl.BlockSpec((BLOCK, NUM_CODES), lambda i: (i, 0)),
            pl.BlockSpec((BLOCK, DIM), lambda i: (i, 0)),
            pl.BlockSpec((BLOCK, DIM), lambda i: (i, 0)),
            pl.BlockSpec((BLOCK, DIM), lambda i: (i, 0)),
            pl.BlockSpec((1, 1), lambda i: (0, 0)),
            pl.BlockSpec((1, 1), lambda i: (0, 0)),
        ],
        scratch_shapes=[
            pltpu.VMEM((1, NUM_CODES), jnp.float32),
            pltpu.VMEM((1, 1), jnp.float32),
        ],
        out_shape=out_shapes,
    )(flat, embedding_weight)

    q0r = q0.reshape(in_shape)
    q1r = q1.reshape(in_shape)
    q2r = q2.reshape(in_shape)
    quantized = jnp.transpose(q0r, (0, 3, 1, 2))
    return (loss[0, 0], quantized, perp[0, 0], enc, (q0r, q1r, q2r))


# speedup vs baseline: 12.1844x; 1.2255x over previous
import functools

import jax
import jax.numpy as jnp
from jax import lax
from jax.experimental import pallas as pl
from jax.experimental.pallas import tpu as pltpu
from jax.experimental.pallas import tpu_sc as plsc

NUM_CODES = 1024
DIM = 64
COMMIT = 0.25
K = 3
ROWS = 16384
BLOCK = 512
GRID = ROWS // BLOCK

_NC, _NS = 2, 16
_NW = _NC * _NS
_B_ALL = K * ROWS
_BPW = _B_ALL // _NW
_CHUNK = 128
_NCHUNK = _BPW // _CHUNK


def _tc_body(x_ref, e_ref, enc_ref, i0_ref, i1_ref, i2_ref, loss_ref,
             perp_ref, cnt_ref, acc_ref):
    i = pl.program_id(0)
    x = x_ref[...]
    e = e_ref[...]

    xsq = jnp.sum(x * x, axis=1, keepdims=True)
    esq = jnp.sum(e * e, axis=1)[None, :]
    xe = jax.lax.dot_general(x, e, (((1,), (1,)), ((), ())),
                             preferred_element_type=jnp.float32)
    d = xsq + esq - 2.0 * xe

    iota = jax.lax.broadcasted_iota(jnp.int32, d.shape, 1)
    idx_refs = (i0_ref, i1_ref, i2_ref)
    for k in range(K):
        dmin = jnp.min(d, axis=1, keepdims=True)
        idx = jnp.min(jnp.where(d == dmin, iota, NUM_CODES), axis=1,
                      keepdims=True)
        idx_refs[k][...] = idx
        if k == 0:
            part_loss = jnp.sum(dmin, keepdims=True)[:1, :1]
        if k < K - 1:
            d = jnp.where(iota == idx, jnp.inf, d)

    last_oh = (iota == idx).astype(jnp.float32)
    enc_ref[...] = last_oh

    @pl.when(i == 0)
    def _():
        acc_ref[...] = jnp.zeros_like(acc_ref)
        cnt_ref[...] = jnp.zeros_like(cnt_ref)

    acc_ref[...] += part_loss
    cnt_ref[...] += jnp.sum(last_oh, axis=0, keepdims=True)

    @pl.when(i == GRID - 1)
    def _():
        loss_ref[...] = acc_ref[...] * (COMMIT / (ROWS * DIM))
        p = cnt_ref[...] * (1.0 / ROWS)
        perp_ref[...] = jnp.exp(-jnp.sum(p * jnp.log(p + 1e-10),
                                         keepdims=True))


@functools.cache
def _sc_gather_fn():
    mesh = plsc.VectorSubcoreMesh(core_axis_name="c", subcore_axis_name="s")

    @functools.partial(
        pl.kernel, mesh=mesh,
        compiler_params=pltpu.CompilerParams(use_tc_tiling_on_sc=False),
        out_type=jax.ShapeDtypeStruct((_B_ALL, DIM), jnp.float32),
        scratch_types=[
            pltpu.VMEM((_BPW,), jnp.int32),
            pltpu.VMEM((_BPW, DIM), jnp.float32),
            pltpu.SemaphoreType.DMA,
        ],
    )
    def _sc_gather(table_hbm, idx_hbm, out_hbm, idx_v, rows_v, sem):
        wid = lax.axis_index("s") * _NC + lax.axis_index("c")
        base = wid * _BPW
        pltpu.sync_copy(idx_hbm.at[pl.ds(base, _BPW)], idx_v)
        copies = []
        for c in range(_NCHUNK):
            copies.append(pltpu.async_copy(
                table_hbm.at[idx_v.at[pl.ds(c * _CHUNK, _CHUNK)]],
                rows_v.at[pl.ds(c * _CHUNK, _CHUNK)], sem))
        for cp in copies:
            cp.wait()
        pltpu.sync_copy(rows_v, out_hbm.at[pl.ds(base, _BPW)])

    return _sc_gather


def kernel(inputs, embedding_weight):
    x = jnp.transpose(inputs, (0, 2, 3, 1))
    in_shape = x.shape
    flat = x.reshape(ROWS, DIM)

    out_shapes = (
        jax.ShapeDtypeStruct((ROWS, NUM_CODES), jnp.float32),
        jax.ShapeDtypeStruct((ROWS, 1), jnp.int32),
        jax.ShapeDtypeStruct((ROWS, 1), jnp.int32),
        jax.ShapeDtypeStruct((ROWS, 1), jnp.int32),
        jax.ShapeDtypeStruct((1, 1), jnp.float32),
        jax.ShapeDtypeStruct((1, 1), jnp.float32),
    )
    enc, i0, i1, i2, loss, perp = pl.pallas_call(
        _tc_body,
        grid=(GRID,),
        in_specs=[
            pl.BlockSpec((BLOCK, DIM), lambda i: (i, 0)),
            pl.BlockSpec((NUM_CODES, DIM), lambda i: (0, 0)),
        ],
        out_specs=[
            pl.BlockSpec((BLOCK, NUM_CODES), lambda i: (i, 0)),
            pl.BlockSpec((BLOCK, 1), lambda i: (i, 0)),
            pl.BlockSpec((BLOCK, 1), lambda i: (i, 0)),
            pl.BlockSpec((BLOCK, 1), lambda i: (i, 0)),
            pl.BlockSpec((1, 1), lambda i: (0, 0)),
            pl.BlockSpec((1, 1), lambda i: (0, 0)),
        ],
        scratch_shapes=[
            pltpu.VMEM((1, NUM_CODES), jnp.float32),
            pltpu.VMEM((1, 1), jnp.float32),
        ],
        out_shape=out_shapes,
    )(flat, embedding_weight)

    idx_all = jnp.concatenate([i0, i1, i2], axis=0)[:, 0]
    q_all = _sc_gather_fn()(embedding_weight, idx_all)
    q = q_all.reshape((K,) + in_shape)
    q0r, q1r, q2r = q[0], q[1], q[2]
    quantized = jnp.transpose(q0r, (0, 3, 1, 2))
    return (loss[0, 0], quantized, perp[0, 0], enc, (q0r, q1r, q2r))


# speedup vs baseline: 13.6898x; 1.1236x over previous
import functools

import jax
import jax.numpy as jnp
from jax import lax
from jax.experimental import pallas as pl
from jax.experimental.pallas import tpu as pltpu
from jax.experimental.pallas import tpu_sc as plsc

NUM_CODES = 1024
DIM = 64
COMMIT = 0.25
K = 3
ROWS = 16384
BLOCK = 512
GRID = ROWS // BLOCK

_NC, _NS = 2, 16
_NW = _NC * _NS
_RPW = ROWS // _NW
_CHUNK = 128
_NCHUNK = _RPW // _CHUNK


def _tc_body(x_ref, e_ref, enc_ref, i0_ref, i1_ref, i2_ref, loss_ref,
             perp_ref, cnt_ref, acc_ref):
    i = pl.program_id(0)
    x = x_ref[...]
    e = e_ref[...]

    xsq = jnp.sum(x * x, axis=1, keepdims=True)
    esq = jnp.sum(e * e, axis=1)[None, :]
    xe = jax.lax.dot_general(x, e, (((1,), (1,)), ((), ())),
                             preferred_element_type=jnp.float32)
    d = xsq + esq - 2.0 * xe

    iota = jax.lax.broadcasted_iota(jnp.int32, d.shape, 1)
    idx_refs = (i0_ref, i1_ref, i2_ref)
    for k in range(K):
        dmin = jnp.min(d, axis=1, keepdims=True)
        idx = jnp.min(jnp.where(d == dmin, iota, NUM_CODES), axis=1,
                      keepdims=True)
        idx_refs[k][...] = idx
        if k == 0:
            part_loss = jnp.sum(dmin, keepdims=True)[:1, :1]
        if k < K - 1:
            d = jnp.where(iota == idx, jnp.inf, d)

    last_oh = (iota == idx).astype(jnp.float32)
    enc_ref[...] = last_oh

    @pl.when(i == 0)
    def _():
        acc_ref[...] = jnp.zeros_like(acc_ref)
        cnt_ref[...] = jnp.zeros_like(cnt_ref)

    acc_ref[...] += part_loss
    cnt_ref[...] += jnp.sum(last_oh, axis=0, keepdims=True)

    @pl.when(i == GRID - 1)
    def _():
        loss_ref[...] = acc_ref[...] * (COMMIT / (ROWS * DIM))
        p = cnt_ref[...] * (1.0 / ROWS)
        perp_ref[...] = jnp.exp(-jnp.sum(p * jnp.log(p + 1e-10),
                                         keepdims=True))


@functools.cache
def _sc_gather_fn():
    mesh = plsc.VectorSubcoreMesh(core_axis_name="c", subcore_axis_name="s")
    row_ty = jax.ShapeDtypeStruct((ROWS, DIM), jnp.float32)

    @functools.partial(
        pl.kernel, mesh=mesh,
        compiler_params=pltpu.CompilerParams(use_tc_tiling_on_sc=False),
        out_type=(row_ty, row_ty, row_ty),
        scratch_types=[
            pltpu.VMEM((_RPW,), jnp.int32),
            pltpu.VMEM((_RPW, DIM), jnp.float32),
            pltpu.SemaphoreType.DMA,
        ],
    )
    def _sc_gather(table_hbm, i0_hbm, i1_hbm, i2_hbm, o0_hbm, o1_hbm, o2_hbm,
                   idx_v, rows_v, sem):
        wid = lax.axis_index("s") * _NC + lax.axis_index("c")
        base = wid * _RPW
        for idx_hbm, out_hbm in ((i0_hbm, o0_hbm), (i1_hbm, o1_hbm),
                                 (i2_hbm, o2_hbm)):
            pltpu.sync_copy(idx_hbm.at[pl.ds(base, _RPW)], idx_v)
            copies = []
            for c in range(_NCHUNK):
                copies.append(pltpu.async_copy(
                    table_hbm.at[idx_v.at[pl.ds(c * _CHUNK, _CHUNK)]],
                    rows_v.at[pl.ds(c * _CHUNK, _CHUNK)], sem))
            for cp in copies:
                cp.wait()
            pltpu.sync_copy(rows_v, out_hbm.at[pl.ds(base, _RPW)])

    return _sc_gather


def kernel(inputs, embedding_weight):
    flat = jnp.transpose(inputs, (0, 2, 3, 1)).reshape(ROWS, DIM)

    out_shapes = (
        jax.ShapeDtypeStruct((ROWS, NUM_CODES), jnp.float32),
        jax.ShapeDtypeStruct((ROWS, 1), jnp.int32),
        jax.ShapeDtypeStruct((ROWS, 1), jnp.int32),
        jax.ShapeDtypeStruct((ROWS, 1), jnp.int32),
        jax.ShapeDtypeStruct((1, 1), jnp.float32),
        jax.ShapeDtypeStruct((1, 1), jnp.float32),
    )
    enc, i0, i1, i2, loss, perp = pl.pallas_call(
        _tc_body,
        grid=(GRID,),
        in_specs=[
            pl.BlockSpec((BLOCK, DIM), lambda i: (i, 0)),
            pl.BlockSpec((NUM_CODES, DIM), lambda i: (0, 0)),
        ],
        out_specs=[
            pl.BlockSpec((BLOCK, NUM_CODES), lambda i: (i, 0)),
            pl.BlockSpec((BLOCK, 1), lambda i: (i, 0)),
            pl.BlockSpec((BLOCK, 1), lambda i: (i, 0)),
            pl.BlockSpec((BLOCK, 1), lambda i: (i, 0)),
            pl.BlockSpec((1, 1), lambda i: (0, 0)),
            pl.BlockSpec((1, 1), lambda i: (0, 0)),
        ],
        scratch_shapes=[
            pltpu.VMEM((1, NUM_CODES), jnp.float32),
            pltpu.VMEM((1, 1), jnp.float32),
        ],
        out_shape=out_shapes,
    )(flat, embedding_weight)

    q0, q1, q2 = _sc_gather_fn()(embedding_weight, i0.reshape(ROWS),
                                 i1.reshape(ROWS), i2.reshape(ROWS))
    in_shape = (16, 32, 32, DIM)
    q0r = q0.reshape(in_shape)
    q1r = q1.reshape(in_shape)
    q2r = q2.reshape(in_shape)
    quantized = jnp.transpose(q0r, (0, 3, 1, 2))
    return (loss[0, 0], quantized, perp[0, 0], enc, (q0r, q1r, q2r))
